# baseline (speedup 1.0000x reference)
import jax
import jax.numpy as jnp
from jax.experimental import pallas as pl
from jax.experimental.pallas import tpu as pltpu

SEQ = 2048
EXPERTS = 16
HIDDEN = 2048
TOPK = 2
CAPACITY = 256
BLK = 512
NBLK = SEQ // BLK
MBLK = 256
NMBLK = SEQ // MBLK


def _routing_body(hs_ref, wg_ref, rp_ref, p0_ref, p1_ref, c0_ref,
                  cnt_ref, laux_ref, rate_ref, offs0, offs1, sumg, tril_s):
    i = pl.program_id(0)

    @pl.when(i == 0)
    def _init():
        offs0[...] = jnp.zeros_like(offs0)
        offs1[...] = jnp.zeros_like(offs1)
        sumg[...] = jnp.zeros_like(sumg)
        rows = jax.lax.broadcasted_iota(jnp.int32, (BLK, BLK), 0)
        cols = jax.lax.broadcasted_iota(jnp.int32, (BLK, BLK), 1)
        tril_s[...] = (cols < rows).astype(jnp.float32)

    x = hs_ref[...]
    w = wg_ref[...]
    logits = jax.lax.dot_general(
        x, w, (((1,), (1,)), ((), ())), preferred_element_type=jnp.float32
    )

    m = jnp.max(logits, axis=1, keepdims=True)
    ex = jnp.exp(logits - m)
    g = ex / jnp.sum(ex, axis=1, keepdims=True)

    erows = jax.lax.broadcasted_iota(jnp.int32, (EXPERTS, EXPERTS), 0)
    ecols = jax.lax.broadcasted_iota(jnp.int32, (EXPERTS, EXPERTS), 1)
    triu = (erows < ecols).astype(jnp.float32)
    v0 = jnp.max(g, axis=1, keepdims=True)
    cand0 = (g == v0).astype(jnp.float32)
    before0 = jax.lax.dot_general(
        cand0, triu, (((1,), (0,)), ((), ())),
        preferred_element_type=jnp.float32,
    )
    m0 = jnp.logical_and(cand0 > 0.0, before0 == 0.0)
    g_ex = jnp.where(m0, -jnp.inf, g)
    v1 = jnp.max(g_ex, axis=1, keepdims=True)
    cand1 = (g_ex == v1).astype(jnp.float32)
    before1 = jax.lax.dot_general(
        cand1, triu, (((1,), (0,)), ((), ())),
        preferred_element_type=jnp.float32,
    )
    m1 = jnp.logical_and(cand1 > 0.0, before1 == 0.0)

    gates_s = jnp.maximum(v0 + v1, jnp.finfo(jnp.float32).eps)
    rp_ref[...] = g / gates_s

    m0f = m0.astype(jnp.float32)
    m1f = m1.astype(jnp.float32)
    tril = tril_s[...]
    exc0 = jax.lax.dot_general(
        tril, m0f, (((1,), (0,)), ((), ())), preferred_element_type=jnp.float32
    )
    exc1 = jax.lax.dot_general(
        tril, m1f, (((1,), (0,)), ((), ())), preferred_element_type=jnp.float32
    )

    p0_ref[...] = jnp.where(m0, offs0[...] + exc0, -1.0)
    p1_ref[...] = jnp.where(m1, offs1[...] + exc1, -1.0)

    tot0 = exc0[BLK - 1 : BLK, :] + m0f[BLK - 1 : BLK, :]
    tot1 = exc1[BLK - 1 : BLK, :] + m1f[BLK - 1 : BLK, :]
    offs0[...] = offs0[...] + tot0
    offs1[...] = offs1[...] + tot1
    sumg[...] = sumg[...] + jnp.sum(g, axis=0, keepdims=True)
    c0_ref[...] = offs0[...]

    @pl.when(i == NBLK - 1)
    def _finish():
        ctot = offs0[...] + offs1[...]
        cnt_ref[...] = ctot.astype(jnp.int32)
        inv_s = 1.0 / SEQ
        laux = (EXPERTS * EXPERTS) * jnp.mean(
            (ctot * inv_s) * (sumg[...] * inv_s)
        )
        laux_ref[0, 0] = laux
        rate_ref[0, 0] = jnp.sum(jnp.minimum(ctot, float(CAPACITY))) / (
            SEQ * TOPK
        )


def _materialize_body(rp_ref, p0_ref, p1_ref, c0_ref, comb_ref, disp_ref):
    rp = rp_ref[...]
    p0 = p0_ref[...]
    p1p = p1_ref[...]
    c0 = c0_ref[...]

    p1 = jnp.where(p1p >= 0.0, p1p + c0, -1.0)
    tp = jnp.maximum(p0, p1)
    valid = jnp.logical_and(tp >= 0.0, tp < float(CAPACITY))
    tpc = jnp.where(valid, tp, -1.0).astype(jnp.int32)

    cap_iota = jax.lax.broadcasted_iota(
        jnp.int32, (MBLK, EXPERTS, CAPACITY), 2
    )
    disp = tpc[:, :, None] == cap_iota
    disp_ref[...] = disp.astype(jnp.int8)
    comb_ref[...] = jnp.where(disp, rp[:, :, None], 0.0)


@jax.jit
def _run(hs, wg):
    meta_spec = pl.BlockSpec((BLK, EXPERTS), lambda i: (i, 0))
    vec_spec = pl.BlockSpec((1, EXPERTS), lambda i: (0, 0))
    smem_spec = pl.BlockSpec(memory_space=pltpu.SMEM)

    rp, p0, p1, c0, cnt, laux, rate = pl.pallas_call(
        _routing_body,
        grid=(NBLK,),
        in_specs=[
            pl.BlockSpec((BLK, HIDDEN), lambda i: (i, 0)),
            pl.BlockSpec((EXPERTS, HIDDEN), lambda i: (0, 0)),
        ],
        out_specs=[meta_spec, meta_spec, meta_spec, vec_spec, vec_spec,
                   smem_spec, smem_spec],
        out_shape=[
            jax.ShapeDtypeStruct((SEQ, EXPERTS), jnp.float32),
            jax.ShapeDtypeStruct((SEQ, EXPERTS), jnp.float32),
            jax.ShapeDtypeStruct((SEQ, EXPERTS), jnp.float32),
            jax.ShapeDtypeStruct((1, EXPERTS), jnp.float32),
            jax.ShapeDtypeStruct((1, EXPERTS), jnp.int32),
            jax.ShapeDtypeStruct((1, 1), jnp.float32),
            jax.ShapeDtypeStruct((1, 1), jnp.float32),
        ],
        scratch_shapes=[
            pltpu.VMEM((1, EXPERTS), jnp.float32),
            pltpu.VMEM((1, EXPERTS), jnp.float32),
            pltpu.VMEM((1, EXPERTS), jnp.float32),
            pltpu.VMEM((BLK, BLK), jnp.float32),
        ],
    )(hs, wg)

    mmeta_spec = pl.BlockSpec((MBLK, EXPERTS), lambda i: (i, 0))
    comb, disp8 = pl.pallas_call(
        _materialize_body,
        grid=(NMBLK,),
        in_specs=[mmeta_spec, mmeta_spec, mmeta_spec, vec_spec],
        out_specs=[
            pl.BlockSpec((MBLK, EXPERTS, CAPACITY), lambda i: (i, 0, 0)),
            pl.BlockSpec((MBLK, EXPERTS, CAPACITY), lambda i: (i, 0, 0)),
        ],
        out_shape=[
            jax.ShapeDtypeStruct((SEQ, EXPERTS, CAPACITY), jnp.float32),
            jax.ShapeDtypeStruct((SEQ, EXPERTS, CAPACITY), jnp.int8),
        ],
    )(rp, p0, p1, c0)

    return (
        laux.reshape(()),
        rate.reshape(()),
        comb,
        disp8.astype(jnp.bool_),
        cnt.reshape(EXPERTS),
    )


def kernel(hidden_states, wg_weight):
    hs = hidden_states.reshape(-1, HIDDEN).astype(jnp.float32)
    return _run(hs, wg_weight)


# speedup vs baseline: 1.0098x; 1.0098x over previous
import jax
import jax.numpy as jnp
from jax.experimental import pallas as pl
from jax.experimental.pallas import tpu as pltpu

SEQ = 2048
EXPERTS = 16
HIDDEN = 2048
TOPK = 2
CAPACITY = 256
BLK = 512
NBLK = SEQ // BLK
MBLK = 512
NMBLK = SEQ // MBLK


def _routing_body(hs_ref, wg_ref, rp_ref, p0_ref, p1_ref, c0_ref,
                  cnt_ref, laux_ref, rate_ref, offs0, offs1, sumg, tril_s):
    i = pl.program_id(0)

    @pl.when(i == 0)
    def _init():
        offs0[...] = jnp.zeros_like(offs0)
        offs1[...] = jnp.zeros_like(offs1)
        sumg[...] = jnp.zeros_like(sumg)
        rows = jax.lax.broadcasted_iota(jnp.int32, (BLK, BLK), 0)
        cols = jax.lax.broadcasted_iota(jnp.int32, (BLK, BLK), 1)
        tril_s[...] = (cols < rows).astype(jnp.float32)

    x = hs_ref[...]
    w = wg_ref[...]
    logits = jax.lax.dot_general(
        x, w, (((1,), (1,)), ((), ())), preferred_element_type=jnp.float32
    )

    m = jnp.max(logits, axis=1, keepdims=True)
    ex = jnp.exp(logits - m)
    g = ex / jnp.sum(ex, axis=1, keepdims=True)

    erows = jax.lax.broadcasted_iota(jnp.int32, (EXPERTS, EXPERTS), 0)
    ecols = jax.lax.broadcasted_iota(jnp.int32, (EXPERTS, EXPERTS), 1)
    triu = (erows < ecols).astype(jnp.float32)
    v0 = jnp.max(g, axis=1, keepdims=True)
    cand0 = (g == v0).astype(jnp.float32)
    before0 = jax.lax.dot_general(
        cand0, triu, (((1,), (0,)), ((), ())),
        preferred_element_type=jnp.float32,
    )
    m0 = jnp.logical_and(cand0 > 0.0, before0 == 0.0)
    g_ex = jnp.where(m0, -jnp.inf, g)
    v1 = jnp.max(g_ex, axis=1, keepdims=True)
    cand1 = (g_ex == v1).astype(jnp.float32)
    before1 = jax.lax.dot_general(
        cand1, triu, (((1,), (0,)), ((), ())),
        preferred_element_type=jnp.float32,
    )
    m1 = jnp.logical_and(cand1 > 0.0, before1 == 0.0)

    gates_s = jnp.maximum(v0 + v1, jnp.finfo(jnp.float32).eps)
    rp_ref[...] = g / gates_s

    m0f = m0.astype(jnp.float32)
    m1f = m1.astype(jnp.float32)
    tril = tril_s[...]
    exc0 = jax.lax.dot_general(
        tril, m0f, (((1,), (0,)), ((), ())), preferred_element_type=jnp.float32
    )
    exc1 = jax.lax.dot_general(
        tril, m1f, (((1,), (0,)), ((), ())), preferred_element_type=jnp.float32
    )

    p0_ref[...] = jnp.where(m0, offs0[...] + exc0, -1.0)
    p1_ref[...] = jnp.where(m1, offs1[...] + exc1, -1.0)

    tot0 = exc0[BLK - 1 : BLK, :] + m0f[BLK - 1 : BLK, :]
    tot1 = exc1[BLK - 1 : BLK, :] + m1f[BLK - 1 : BLK, :]
    offs0[...] = offs0[...] + tot0
    offs1[...] = offs1[...] + tot1
    sumg[...] = sumg[...] + jnp.sum(g, axis=0, keepdims=True)
    c0_ref[...] = offs0[...]

    @pl.when(i == NBLK - 1)
    def _finish():
        ctot = offs0[...] + offs1[...]
        cnt_ref[...] = ctot.astype(jnp.int32)
        inv_s = 1.0 / SEQ
        laux = (EXPERTS * EXPERTS) * jnp.mean(
            (ctot * inv_s) * (sumg[...] * inv_s)
        )
        laux_ref[0, 0] = laux
        rate_ref[0, 0] = jnp.sum(jnp.minimum(ctot, float(CAPACITY))) / (
            SEQ * TOPK
        )


def _materialize_body(rp_ref, p0_ref, p1_ref, c0_ref, comb_ref, disp_ref):
    rp = rp_ref[...]
    p0 = p0_ref[...]
    p1p = p1_ref[...]
    c0 = c0_ref[...]

    p1 = jnp.where(p1p >= 0.0, p1p + c0, -1.0)
    tp = jnp.maximum(p0, p1)
    valid = jnp.logical_and(tp >= 0.0, tp < float(CAPACITY))
    tpc = jnp.where(valid, tp, -1.0).astype(jnp.int32)

    cap_iota = jax.lax.broadcasted_iota(
        jnp.int32, (MBLK, EXPERTS, CAPACITY), 2
    )
    disp = tpc[:, :, None] == cap_iota
    disp_ref[...] = disp.astype(jnp.int8)
    comb_ref[...] = jnp.where(disp, rp[:, :, None], 0.0)


@jax.jit
def _run(hs, wg):
    meta_spec = pl.BlockSpec((BLK, EXPERTS), lambda i: (i, 0))
    vec_spec = pl.BlockSpec((1, EXPERTS), lambda i: (0, 0))
    smem_spec = pl.BlockSpec(memory_space=pltpu.SMEM)

    rp, p0, p1, c0, cnt, laux, rate = pl.pallas_call(
        _routing_body,
        grid=(NBLK,),
        in_specs=[
            pl.BlockSpec((BLK, HIDDEN), lambda i: (i, 0)),
            pl.BlockSpec((EXPERTS, HIDDEN), lambda i: (0, 0)),
        ],
        out_specs=[meta_spec, meta_spec, meta_spec, vec_spec, vec_spec,
                   smem_spec, smem_spec],
        out_shape=[
            jax.ShapeDtypeStruct((SEQ, EXPERTS), jnp.float32),
            jax.ShapeDtypeStruct((SEQ, EXPERTS), jnp.float32),
            jax.ShapeDtypeStruct((SEQ, EXPERTS), jnp.float32),
            jax.ShapeDtypeStruct((1, EXPERTS), jnp.float32),
            jax.ShapeDtypeStruct((1, EXPERTS), jnp.int32),
            jax.ShapeDtypeStruct((1, 1), jnp.float32),
            jax.ShapeDtypeStruct((1, 1), jnp.float32),
        ],
        scratch_shapes=[
            pltpu.VMEM((1, EXPERTS), jnp.float32),
            pltpu.VMEM((1, EXPERTS), jnp.float32),
            pltpu.VMEM((1, EXPERTS), jnp.float32),
            pltpu.VMEM((BLK, BLK), jnp.float32),
        ],
    )(hs, wg)

    mmeta_spec = pl.BlockSpec((MBLK, EXPERTS), lambda i: (i, 0))
    comb, disp8 = pl.pallas_call(
        _materialize_body,
        grid=(NMBLK,),
        in_specs=[mmeta_spec, mmeta_spec, mmeta_spec, vec_spec],
        out_specs=[
            pl.BlockSpec((MBLK, EXPERTS, CAPACITY), lambda i: (i, 0, 0)),
            pl.BlockSpec((MBLK, EXPERTS, CAPACITY), lambda i: (i, 0, 0)),
        ],
        out_shape=[
            jax.ShapeDtypeStruct((SEQ, EXPERTS, CAPACITY), jnp.float32),
            jax.ShapeDtypeStruct((SEQ, EXPERTS, CAPACITY), jnp.int8),
        ],
    )(rp, p0, p1, c0)

    return (
        laux.reshape(()),
        rate.reshape(()),
        comb,
        disp8.astype(jnp.bool_),
        cnt.reshape(EXPERTS),
    )


def kernel(hidden_states, wg_weight):
    hs = hidden_states.reshape(-1, HIDDEN).astype(jnp.float32)
    return _run(hs, wg_weight)
